# baseline (speedup 1.0000x reference)
import jax
import jax.numpy as jnp
from jax import lax
from jax.experimental import pallas as pl
from jax.experimental.pallas import tpu as pltpu
from jax.experimental.pallas import tpu_sc as plsc

D = 1001000
LAT = 64
W = 384
NG = W // 16
DPAD = 1001088
NCHUNK = DPAD // W
NC = 2
NS = 16
NW = NC * NS


def _sc_body(zt_hbm, theta_hbm, bT_hbm, out_hbm,
             zt_v, b0_v, b1_v, a0_v, a1_v, bsem0, bsem1, osem0, osem1):
    wid = lax.axis_index("s") * NC + lax.axis_index("c")
    n_mine = NCHUNK // NW + jnp.where(wid < NCHUNK % NW, 1, 0)

    pltpu.sync_copy(zt_hbm, zt_v)

    bufs = ((b0_v, a0_v, bsem0, osem0), (b1_v, a1_v, bsem1, osem1))

    def chunk_col0(j):
        return pl.multiple_of((wid + j * NW) * W, 128)

    def start_in(j, b):
        col0 = chunk_col0(j)
        bv, av, bsem, _ = bufs[b]
        pltpu.async_copy(bT_hbm.at[:, pl.ds(col0, W)], bv, bsem)
        pltpu.async_copy(theta_hbm.at[pl.ds(col0, W)], av, bsem)

    def wait_in(b):
        bv, av, bsem, _ = bufs[b]
        pltpu.make_async_copy(bT_hbm.at[:, pl.ds(0, W)], bv, bsem).wait()
        pltpu.make_async_copy(theta_hbm.at[pl.ds(0, W)], av, bsem).wait()

    def start_out(j, b):
        col0 = chunk_col0(j)
        _, av, _, osem = bufs[b]
        pltpu.async_copy(av, out_hbm.at[pl.ds(col0, W)], osem)

    def wait_out(b):
        _, av, _, osem = bufs[b]
        pltpu.make_async_copy(av, out_hbm.at[pl.ds(0, W)], osem).wait()

    def compute(b):
        bv, av, _, _ = bufs[b]
        accs0 = tuple(av[pl.ds(g * 16, 16)] for g in range(NG))

        def do_k(k, accs):
            zb = zt_v[pl.ds(pl.multiple_of(k * 16, 16), 16)]
            return tuple(accs[g] + bv[k, pl.ds(g * 16, 16)] * zb
                         for g in range(NG))

        accs = lax.fori_loop(0, LAT, do_k, accs0, unroll=4)
        for g in range(NG):
            av[pl.ds(g * 16, 16)] = accs[g]

    @pl.when(n_mine > 0)
    def _prologue():
        start_in(0, 0)

    def pair_body(p, carry):
        for b in (0, 1):
            j = 2 * p + b

            @pl.when(j < n_mine)
            def _step():
                @pl.when(j + 1 < n_mine)
                def _prefetch():
                    @pl.when(j >= 1)
                    def _drain():
                        wait_out(1 - b)
                    start_in(j + 1, 1 - b)

                wait_in(b)
                compute(b)
                start_out(j, b)
        return carry

    lax.fori_loop(0, (n_mine + 1) // 2, pair_body, 0)

    last = n_mine - 1

    @pl.when(n_mine > 0)
    def _final_drain():
        @pl.when(last % 2 == 0)
        def _d0():
            wait_out(0)

        @pl.when(last % 2 == 1)
        def _d1():
            wait_out(1)


@jax.jit
def _projection(zt, theta_base, bT):
    mesh = plsc.VectorSubcoreMesh(core_axis_name="c", subcore_axis_name="s",
                                  num_cores=NC, num_subcores=NS)
    return pl.kernel(
        _sc_body,
        out_type=jax.ShapeDtypeStruct((D,), jnp.float32),
        mesh=mesh,
        scratch_types=[
            pltpu.VMEM((LAT * 16,), jnp.float32),
            pltpu.VMEM((LAT, W), jnp.float32),
            pltpu.VMEM((LAT, W), jnp.float32),
            pltpu.VMEM((W,), jnp.float32),
            pltpu.VMEM((W,), jnp.float32),
            pltpu.SemaphoreType.DMA,
            pltpu.SemaphoreType.DMA,
            pltpu.SemaphoreType.DMA,
            pltpu.SemaphoreType.DMA,
        ],
        compiler_params=pltpu.CompilerParams(needs_layout_passes=False),
    )(zt, theta_base, bT)


def kernel(z, theta_base, basis_matrix):
    zt = jnp.broadcast_to(z[:, None], (LAT, 16)).reshape(-1)
    return _projection(zt, theta_base, basis_matrix.T)
